# baseline (speedup 1.0000x reference)
import jax
import jax.numpy as jnp
from jax import lax
from jax.experimental import pallas as pl
from jax.experimental.pallas import tpu as pltpu
from jax.experimental.pallas import tpu_sc as plsc

VOCAB = 100000
CAT = 1000
MAXPOS = 200
EMB = 128
HID = 512
B = 1024
S = 200
N = B * S
EPS = 1e-12

NC = 2
NS = 16
NW = NC * NS
CHUNK = 128
TOK_PER_W = N // NW
N_CHUNKS = TOK_PER_W // CHUNK

TB = 1024


PK = EMB // 2


def _sc_gather_body(ids_hbm, id_tab, o1, idx1, r0, r1, sem0, sem1):
    wid = lax.axis_index("s") * NC + lax.axis_index("c")
    wbase = wid * TOK_PER_W
    bufs = (r0, r1)
    sems = (sem0, sem1)

    pltpu.sync_copy(ids_hbm.at[pl.ds(wbase, TOK_PER_W)], idx1)

    def fire(c, s):
        pltpu.async_copy(id_tab.at[idx1.at[pl.ds(c * CHUNK, CHUNK)]],
                         bufs[s], sems[s])

    def drain(s):
        pltpu.make_async_copy(o1.at[pl.ds(0, CHUNK)], bufs[s], sems[s]).wait()

    def writeback(c, s):
        pltpu.sync_copy(bufs[s], o1.at[pl.ds(wbase + c * CHUNK, CHUNK)])

    fire(0, 0)
    fire(1, 1)

    def pair(p, _):
        c = 2 * p
        drain(0)
        writeback(c, 0)
        fire(c + 2, 0)
        drain(1)
        writeback(c + 1, 1)
        fire(c + 3, 1)
        return _

    lax.fori_loop(0, N_CHUNKS // 2 - 1, pair, None)
    drain(0)
    writeback(N_CHUNKS - 2, 0)
    drain(1)
    writeback(N_CHUNKS - 1, 1)


def _sc_gather(ids, id_tab):
    mesh = plsc.VectorSubcoreMesh(core_axis_name="c", subcore_axis_name="s")
    f = pl.kernel(
        _sc_gather_body,
        out_type=jax.ShapeDtypeStruct((N, PK), jnp.float32),
        mesh=mesh,
        scratch_types=[
            pltpu.VMEM((TOK_PER_W,), jnp.int32),
            pltpu.VMEM((CHUNK, PK), jnp.float32),
            pltpu.VMEM((CHUNK, PK), jnp.float32),
            pltpu.SemaphoreType.DMA,
            pltpu.SemaphoreType.DMA,
        ],
        compiler_params=pltpu.CompilerParams(use_tc_tiling_on_sc=False),
    )
    return f(ids, id_tab)


def _tc_body(x1, catb, posb, ct, pt, w1, w2, w3, bb, lw, lb, o):
    cat = catb[0, 0, :].reshape(TB, 1)
    pos = posb[0, 0, :].reshape(TB, 1)
    oh_c = (lax.broadcasted_iota(jnp.int32, (TB, CAT), 1) == cat).astype(jnp.bfloat16)
    oh_p = (lax.broadcasted_iota(jnp.int32, (TB, MAXPOS), 1) == pos).astype(jnp.bfloat16)
    x2 = jnp.dot(oh_c, ct[...],
                 preferred_element_type=jnp.float32).astype(jnp.bfloat16)
    x3 = jnp.dot(oh_p, pt[...],
                 preferred_element_type=jnp.float32).astype(jnp.bfloat16)
    acc = jnp.dot(x1[...], w1[...], preferred_element_type=jnp.float32)
    acc += jnp.dot(x2, w2[...], preferred_element_type=jnp.float32)
    acc += jnp.dot(x3, w3[...], preferred_element_type=jnp.float32)
    acc += bb[...]
    m = jnp.mean(acc, axis=-1, keepdims=True)
    d = acc - m
    v = jnp.mean(d * d, axis=-1, keepdims=True)
    o[...] = d * lax.rsqrt(v + EPS) * lw[...] + lb[...]


def _tc_linear_ln(x1, cats, poss, cat_tab, pos_tab, Wt, b, ln_w, ln_b):
    Wtb = Wt.astype(jnp.bfloat16)
    grid = (N // TB,)
    xspec = pl.BlockSpec((TB, EMB), lambda i: (i, 0))
    ispec = pl.BlockSpec((1, 1, TB), lambda i: (i, 0, 0))
    wspec = pl.BlockSpec((EMB, HID), lambda i: (0, 0))
    vspec = pl.BlockSpec((1, HID), lambda i: (0, 0))
    return pl.pallas_call(
        _tc_body,
        grid=grid,
        in_specs=[
            xspec, ispec, ispec,
            pl.BlockSpec((CAT, EMB), lambda i: (0, 0)),
            pl.BlockSpec((MAXPOS, EMB), lambda i: (0, 0)),
            wspec, wspec, wspec, vspec, vspec, vspec,
        ],
        out_specs=pl.BlockSpec((TB, HID), lambda i: (i, 0)),
        out_shape=jax.ShapeDtypeStruct((N, HID), jnp.float32),
    )(x1, cats.reshape(N // TB, 1, TB), poss.reshape(N // TB, 1, TB),
      cat_tab.astype(jnp.bfloat16), pos_tab.astype(jnp.bfloat16),
      Wtb[:EMB], Wtb[EMB:2 * EMB], Wtb[2 * EMB:], b.reshape(1, HID),
      ln_w.reshape(1, HID), ln_b.reshape(1, HID))


def kernel(input_ids, category_ids, position_ids, id_table, cat_table,
           pos_table, W, b, ln_w, ln_b):
    ids = input_ids.reshape(-1).astype(jnp.int32)
    cats = category_ids.reshape(-1).astype(jnp.int32)
    poss = position_ids.reshape(-1).astype(jnp.int32)
    idtab_pk = jax.lax.bitcast_convert_type(
        id_table.astype(jnp.bfloat16).reshape(VOCAB, PK, 2), jnp.float32)
    e1p = _sc_gather(ids, idtab_pk)
    x1 = jax.lax.bitcast_convert_type(e1p, jnp.bfloat16).reshape(N, EMB)
    out = _tc_linear_ln(x1, cats, poss, cat_table, pos_table, W.T, b, ln_w, ln_b)
    return out.reshape(B, S, HID)


# speedup vs baseline: 2.4403x; 2.4403x over previous
import jax
import jax.numpy as jnp
from jax import lax
from jax.experimental import pallas as pl
from jax.experimental.pallas import tpu as pltpu
from jax.experimental.pallas import tpu_sc as plsc

VOCAB = 100000
CAT = 1000
MAXPOS = 200
EMB = 128
HID = 512
B = 1024
S = 200
N = B * S
EPS = 1e-12

NC = 2
NS = 16
NW = NC * NS
CHUNK = 128
TOK_PER_W = N // NW
N_CHUNKS = TOK_PER_W // CHUNK

TB = 2048


def _sc_gather_body(ids_hbm, id_tab, o1, idx1, r0, r1, sem0, sem1):
    wid = lax.axis_index("s") * NC + lax.axis_index("c")
    wbase = wid * TOK_PER_W
    bufs = (r0, r1)
    sems = (sem0, sem1)

    pltpu.sync_copy(ids_hbm.at[pl.ds(wbase, TOK_PER_W)], idx1)

    def fire(c, s):
        pltpu.async_copy(id_tab.at[idx1.at[pl.ds(c * CHUNK, CHUNK)]],
                         bufs[s], sems[s])

    def drain(s):
        pltpu.make_async_copy(o1.at[pl.ds(0, CHUNK)], bufs[s], sems[s]).wait()

    def writeback(c, s):
        pltpu.sync_copy(bufs[s], o1.at[pl.ds(wbase + c * CHUNK, CHUNK)])

    fire(0, 0)
    fire(1, 1)

    def pair(p, _):
        c = 2 * p
        drain(0)
        writeback(c, 0)
        fire(c + 2, 0)
        drain(1)
        writeback(c + 1, 1)
        fire(c + 3, 1)
        return _

    lax.fori_loop(0, N_CHUNKS // 2 - 1, pair, None)
    drain(0)
    writeback(N_CHUNKS - 2, 0)
    drain(1)
    writeback(N_CHUNKS - 1, 1)


def _sc_gather(ids, id_tab):
    mesh = plsc.VectorSubcoreMesh(core_axis_name="c", subcore_axis_name="s")
    f = pl.kernel(
        _sc_gather_body,
        out_type=jax.ShapeDtypeStruct((N, EMB), jnp.float32),
        mesh=mesh,
        scratch_types=[
            pltpu.VMEM((TOK_PER_W,), jnp.int32),
            pltpu.VMEM((CHUNK, EMB), jnp.float32),
            pltpu.VMEM((CHUNK, EMB), jnp.float32),
            pltpu.SemaphoreType.DMA,
            pltpu.SemaphoreType.DMA,
        ],
    )
    return f(ids, id_tab)


def _tc_body(x1, catb, posb, ct, pt, w1, w2, w3, bb, lw, lb, o):
    cat = catb[0, 0, :].reshape(TB, 1)
    pos = posb[0, 0, :].reshape(TB, 1)
    oh_c = (lax.broadcasted_iota(jnp.int32, (TB, CAT), 1) == cat).astype(jnp.bfloat16)
    oh_p = (lax.broadcasted_iota(jnp.int32, (TB, MAXPOS), 1) == pos).astype(jnp.bfloat16)
    x2 = jnp.dot(oh_c, ct[...],
                 preferred_element_type=jnp.float32).astype(jnp.bfloat16)
    x3 = jnp.dot(oh_p, pt[...],
                 preferred_element_type=jnp.float32).astype(jnp.bfloat16)
    acc = jnp.dot(x1[...].astype(jnp.bfloat16), w1[...],
                  preferred_element_type=jnp.float32)
    acc += jnp.dot(x2, w2[...], preferred_element_type=jnp.float32)
    acc += jnp.dot(x3, w3[...], preferred_element_type=jnp.float32)
    acc += bb[...]
    m = jnp.mean(acc, axis=-1, keepdims=True)
    d = acc - m
    v = jnp.mean(d * d, axis=-1, keepdims=True)
    o[...] = d * lax.rsqrt(v + EPS) * lw[...] + lb[...]


def _tc_linear_ln(x1, cats, poss, cat_tab, pos_tab, Wt, b, ln_w, ln_b):
    Wtb = Wt.astype(jnp.bfloat16)
    grid = (N // TB,)
    xspec = pl.BlockSpec((TB, EMB), lambda i: (i, 0))
    ispec = pl.BlockSpec((1, 1, TB), lambda i: (i, 0, 0))
    wspec = pl.BlockSpec((EMB, HID), lambda i: (0, 0))
    vspec = pl.BlockSpec((1, HID), lambda i: (0, 0))
    return pl.pallas_call(
        _tc_body,
        grid=grid,
        in_specs=[
            xspec, ispec, ispec,
            pl.BlockSpec((CAT, EMB), lambda i: (0, 0)),
            pl.BlockSpec((MAXPOS, EMB), lambda i: (0, 0)),
            wspec, wspec, wspec, vspec, vspec, vspec,
        ],
        out_specs=pl.BlockSpec((TB, HID), lambda i: (i, 0)),
        out_shape=jax.ShapeDtypeStruct((N, HID), jnp.float32),
    )(x1, cats.reshape(N // TB, 1, TB), poss.reshape(N // TB, 1, TB),
      cat_tab.astype(jnp.bfloat16), pos_tab.astype(jnp.bfloat16),
      Wtb[:EMB], Wtb[EMB:2 * EMB], Wtb[2 * EMB:], b.reshape(1, HID),
      ln_w.reshape(1, HID), ln_b.reshape(1, HID))


def kernel(input_ids, category_ids, position_ids, id_table, cat_table,
           pos_table, W, b, ln_w, ln_b):
    ids = input_ids.reshape(-1).astype(jnp.int32)
    cats = category_ids.reshape(-1).astype(jnp.int32)
    poss = position_ids.reshape(-1).astype(jnp.int32)
    e1 = _sc_gather(ids, id_table)
    out = _tc_linear_ln(e1, cats, poss, cat_table, pos_table, W.T, b, ln_w, ln_b)
    return out.reshape(B, S, HID)


# speedup vs baseline: 2.7147x; 1.1124x over previous
import jax
import jax.numpy as jnp
from jax import lax
from jax.experimental import pallas as pl
from jax.experimental.pallas import tpu as pltpu
from jax.experimental.pallas import tpu_sc as plsc

VOCAB = 100000
CAT = 1000
MAXPOS = 200
EMB = 128
HID = 512
B = 1024
S = 200
N = B * S
EPS = 1e-12

NC = 2
NS = 16
NW = NC * NS
CHUNK = 128
TOK_PER_W = N // NW
N_CHUNKS = TOK_PER_W // CHUNK

TB = 4096


def _sc_gather_body(ids_hbm, id_tab, o1, idx1, r0, r1, sem0, sem1):
    wid = lax.axis_index("s") * NC + lax.axis_index("c")
    wbase = wid * TOK_PER_W
    bufs = (r0, r1)
    sems = (sem0, sem1)

    pltpu.sync_copy(ids_hbm.at[pl.ds(wbase, TOK_PER_W)], idx1)

    def fire(c, s):
        pltpu.async_copy(id_tab.at[idx1.at[pl.ds(c * CHUNK, CHUNK)]],
                         bufs[s], sems[s])

    def drain(s):
        pltpu.make_async_copy(o1.at[pl.ds(0, CHUNK)], bufs[s], sems[s]).wait()

    def writeback(c, s):
        pltpu.sync_copy(bufs[s], o1.at[pl.ds(wbase + c * CHUNK, CHUNK)])

    fire(0, 0)
    fire(1, 1)

    def pair(p, _):
        c = 2 * p
        drain(0)
        writeback(c, 0)
        fire(c + 2, 0)
        drain(1)
        writeback(c + 1, 1)
        fire(c + 3, 1)
        return _

    lax.fori_loop(0, N_CHUNKS // 2 - 1, pair, None)
    drain(0)
    writeback(N_CHUNKS - 2, 0)
    drain(1)
    writeback(N_CHUNKS - 1, 1)


def _sc_gather(ids, id_tab):
    mesh = plsc.VectorSubcoreMesh(core_axis_name="c", subcore_axis_name="s")
    f = pl.kernel(
        _sc_gather_body,
        out_type=jax.ShapeDtypeStruct((N, EMB), jnp.float32),
        mesh=mesh,
        scratch_types=[
            pltpu.VMEM((TOK_PER_W,), jnp.int32),
            pltpu.VMEM((CHUNK, EMB), jnp.float32),
            pltpu.VMEM((CHUNK, EMB), jnp.float32),
            pltpu.SemaphoreType.DMA,
            pltpu.SemaphoreType.DMA,
        ],
    )
    return f(ids, id_tab)


def _tc_body(x1, catb, posb, ct, pt, w1, w2, w3, bb, lw, lb, o):
    cat = catb[0, 0, :].reshape(TB, 1)
    pos = posb[0, 0, :].reshape(TB, 1)
    oh_c = (lax.broadcasted_iota(jnp.int32, (TB, CAT), 1) == cat).astype(jnp.bfloat16)
    oh_p = (lax.broadcasted_iota(jnp.int32, (TB, MAXPOS), 1) == pos).astype(jnp.bfloat16)
    x2 = jnp.dot(oh_c, ct[...],
                 preferred_element_type=jnp.float32).astype(jnp.bfloat16)
    x3 = jnp.dot(oh_p, pt[...],
                 preferred_element_type=jnp.float32).astype(jnp.bfloat16)
    acc = jnp.dot(x1[...], w1[...], preferred_element_type=jnp.float32)
    acc += jnp.dot(x2, w2[...], preferred_element_type=jnp.float32)
    acc += jnp.dot(x3, w3[...], preferred_element_type=jnp.float32)
    acc += bb[...]
    m = jnp.mean(acc, axis=-1, keepdims=True)
    d = acc - m
    v = jnp.mean(d * d, axis=-1, keepdims=True)
    o[...] = d * (lax.rsqrt(v + EPS) * lw[...]) + lb[...]


def _tc_linear_ln(x1, cats, poss, cat_tab, pos_tab, Wt, b, ln_w, ln_b):
    Wtb = Wt.astype(jnp.bfloat16)
    grid = (N // TB,)
    xspec = pl.BlockSpec((TB, EMB), lambda i: (i, 0))
    ispec = pl.BlockSpec((1, 1, TB), lambda i: (i, 0, 0))
    wspec = pl.BlockSpec((EMB, HID), lambda i: (0, 0))
    vspec = pl.BlockSpec((1, HID), lambda i: (0, 0))
    return pl.pallas_call(
        _tc_body,
        grid=grid,
        in_specs=[
            xspec, ispec, ispec,
            pl.BlockSpec((CAT, EMB), lambda i: (0, 0)),
            pl.BlockSpec((MAXPOS, EMB), lambda i: (0, 0)),
            wspec, wspec, wspec, vspec, vspec, vspec,
        ],
        out_specs=pl.BlockSpec((TB, HID), lambda i: (i, 0)),
        out_shape=jax.ShapeDtypeStruct((N, HID), jnp.float32),
    )(x1, cats.reshape(N // TB, 1, TB), poss.reshape(N // TB, 1, TB),
      cat_tab.astype(jnp.bfloat16), pos_tab.astype(jnp.bfloat16),
      Wt[:EMB], Wtb[EMB:2 * EMB], Wtb[2 * EMB:], b.reshape(1, HID),
      ln_w.reshape(1, HID), ln_b.reshape(1, HID))


def kernel(input_ids, category_ids, position_ids, id_table, cat_table,
           pos_table, W, b, ln_w, ln_b):
    ids = input_ids.reshape(-1).astype(jnp.int32)
    cats = category_ids.reshape(-1).astype(jnp.int32)
    poss = position_ids.reshape(-1).astype(jnp.int32)
    e1 = _sc_gather(ids, id_table)
    out = _tc_linear_ln(e1, cats, poss, cat_table, pos_table, W.T, b, ln_w, ln_b)
    return out.reshape(B, S, HID)
